# Initial kernel scaffold; baseline (speedup 1.0000x reference)
import jax
import jax.numpy as jnp
from jax.experimental import pallas as pl


def kernel(coords, tidx, rs):
    raise NotImplementedError("write your pallas kernel here")


# speedup vs baseline: 6.1648x; 6.1648x over previous
import jax
import jax.numpy as jnp
from jax import lax
from jax.experimental import pallas as pl
from jax.experimental.pallas import tpu as pltpu

_NSEG = 8
_NCLS = 48
_E = 2.718281828459045


def _dot(a, b, dims):
    return lax.dot_general(a, b, dimension_numbers=(dims, ((), ())),
                           preferred_element_type=jnp.float32,
                           precision=lax.Precision.HIGHEST)


def _loss_body(rs_ref, x_ref, lab_ref, out_ref):
    n_pts = x_ref.shape[1]
    col = lax.broadcasted_iota(jnp.int32, (1, n_pts), 1)
    segid = jnp.zeros((1, n_pts), jnp.int32)
    for s in range(1, _NSEG):
        segid += (col >= rs_ref[s]).astype(jnp.int32)

    seg1h = (lax.broadcasted_iota(jnp.int32, (_NSEG, n_pts), 0)
             == segid).astype(jnp.float32)
    labels = lab_ref[0:1, :]
    lab1h = (lax.broadcasted_iota(jnp.int32, (_NCLS, n_pts), 0)
             == labels).astype(jnp.float32)

    x = x_ref[...]

    counts = _dot(seg1h, lab1h, ((1,), (1,)))
    sums = [_dot(seg1h * x[d:d + 1, :], lab1h, ((1,), (1,)))
            for d in range(3)]
    n_s = jnp.sum(seg1h, axis=1, keepdims=True)

    cnt_safe = jnp.where(counts == 0.0, 1.0, counts)
    means = [jnp.where(counts == 0.0, 0.0, sums[d] / cnt_safe) for d in range(3)]

    dist2 = jnp.zeros((_NCLS, n_pts), jnp.float32)
    for d in range(3):
        mrow = _dot(means[d], seg1h, ((0,), (0,)))
        diff = mrow - x[d:d + 1, :]
        dist2 += diff * diff

    logterm = jnp.log(_E * dist2 + 1.0) * lab1h
    distsum = _dot(seg1h, logterm, ((1,), (1,)))

    w = 1.0 - 0.9 * (labels < 0).astype(jnp.float32)
    repterm = jnp.exp(-dist2) * (1.0 - lab1h) * w
    repnum = _dot(seg1h, repterm, ((1,), (1,)))

    present = counts > 0.0
    k_s = jnp.sum(present.astype(jnp.float32), axis=1, keepdims=True)

    dl_c = jnp.where(present, distsum / cnt_safe, 0.0)
    dl_s = jnp.sum(dl_c, axis=1, keepdims=True)
    k_safe = jnp.where(k_s == 0.0, 1.0, k_s)
    distloss_s = jnp.where(k_s == 0.0, 0.0, dl_s / k_safe)

    denom_safe = jnp.where(present, n_s - counts, 1.0)
    rep_c = jnp.where(present, repnum / denom_safe, 0.0)
    reploss_s = jnp.sum(rep_c, axis=1, keepdims=True) / (k_s + 0.001)

    seg_loss = distloss_s + reploss_s
    valid = (n_s >= 20.0) & (k_s > 0.0)
    total = jnp.sum(jnp.where(valid, seg_loss, 0.0), keepdims=True)
    out_ref[...] = total.reshape(1, 1)


def _loss_call(x_pad, lab_pad, rs):
    return pl.pallas_call(
        _loss_body,
        out_shape=jax.ShapeDtypeStruct((1, 1), jnp.float32),
        in_specs=[
            pl.BlockSpec(memory_space=pltpu.SMEM),
            pl.BlockSpec(memory_space=pltpu.VMEM),
            pl.BlockSpec(memory_space=pltpu.VMEM),
        ],
        out_specs=pl.BlockSpec(memory_space=pltpu.VMEM),
    )(rs, x_pad, lab_pad)


@jax.jit
def kernel(coords, tidx, rs):
    n_pts = coords.shape[0]
    x_pad = jnp.concatenate(
        [coords.T, jnp.zeros((5, n_pts), jnp.float32)], axis=0)
    lab_pad = jnp.concatenate(
        [tidx.T, jnp.zeros((7, n_pts), jnp.int32)], axis=0)
    loss = _loss_call(x_pad, lab_pad, rs)
    return (coords, loss[0, 0])


# speedup vs baseline: 26.4736x; 4.2943x over previous
import jax
import jax.numpy as jnp
from jax import lax
from jax.experimental import pallas as pl
from jax.experimental.pallas import tpu as pltpu

_NSEG = 8
_NCLS = 48
_E = 2.718281828459045


def _dot(a, b, dims):
    return lax.dot_general(a, b, dimension_numbers=(dims, ((), ())),
                           preferred_element_type=jnp.float32,
                           precision=lax.Precision.DEFAULT)


def _loss_body(rs_ref, x_ref, lab_ref, out_ref):
    n_pts = x_ref.shape[1]
    col = lax.broadcasted_iota(jnp.int32, (1, n_pts), 1)
    segid = jnp.zeros((1, n_pts), jnp.int32)
    for s in range(1, _NSEG):
        segid += (col >= rs_ref[s]).astype(jnp.int32)

    seg1h = (lax.broadcasted_iota(jnp.int32, (_NSEG, n_pts), 0)
             == segid).astype(jnp.float32)
    labels = lab_ref[0:1, :]
    lab1h = (lax.broadcasted_iota(jnp.int32, (_NCLS, n_pts), 0)
             == labels).astype(jnp.float32)

    x = x_ref[...]

    sx = jnp.concatenate([seg1h * x[d:d + 1, :] for d in range(3)], axis=0)
    stack1 = jnp.concatenate([seg1h, sx], axis=0)
    big1 = _dot(stack1, lab1h, ((1,), (1,)))
    counts = big1[0:_NSEG]
    n_s = jnp.sum(seg1h, axis=1, keepdims=True)

    cnt_safe = jnp.where(counts == 0.0, 1.0, counts)
    cnt3 = jnp.concatenate([counts] * 3, axis=0)
    means = jnp.where(cnt3 == 0.0, 0.0,
                      big1[_NSEG:] / jnp.where(cnt3 == 0.0, 1.0, cnt3))
    msq = (means[0:8] * means[0:8] + means[8:16] * means[8:16]
           + means[16:24] * means[16:24])

    xm = _dot(means, sx, ((0,), (0,)))
    msqrow = _dot(msq, seg1h, ((0,), (0,)))
    xsq = (x[0:1] * x[0:1] + x[1:2] * x[1:2] + x[2:3] * x[2:3])

    dist2 = msqrow + (xsq - 2.0 * xm)
    expd = jnp.exp(-dist2)

    d_own = jnp.sum(dist2 * lab1h, axis=0, keepdims=True)
    lt = jnp.log(_E * d_own + 1.0)
    w = 1.0 - 0.9 * (labels < 0).astype(jnp.float32)
    eo = jnp.exp(-d_own) * w

    stack2 = jnp.concatenate([seg1h * lt, seg1h * eo], axis=0)
    big2 = _dot(stack2, lab1h, ((1,), (1,)))
    distsum = big2[0:_NSEG]
    repown = big2[_NSEG:]

    repall = _dot(seg1h * w, expd, ((1,), (1,)))
    repnum = repall - repown

    present = counts > 0.0
    k_s = jnp.sum(present.astype(jnp.float32), axis=1, keepdims=True)

    dl_c = jnp.where(present, distsum / cnt_safe, 0.0)
    dl_s = jnp.sum(dl_c, axis=1, keepdims=True)
    k_safe = jnp.where(k_s == 0.0, 1.0, k_s)
    distloss_s = jnp.where(k_s == 0.0, 0.0, dl_s / k_safe)

    denom_safe = jnp.where(present, n_s - counts, 1.0)
    rep_c = jnp.where(present, repnum / denom_safe, 0.0)
    reploss_s = jnp.sum(rep_c, axis=1, keepdims=True) / (k_s + 0.001)

    seg_loss = distloss_s + reploss_s
    valid = (n_s >= 20.0) & (k_s > 0.0)
    total = jnp.sum(jnp.where(valid, seg_loss, 0.0), keepdims=True)
    out_ref[...] = total.reshape(1, 1)


def _loss_call(x_t, lab_t, rs):
    return pl.pallas_call(
        _loss_body,
        out_shape=jax.ShapeDtypeStruct((1, 1), jnp.float32),
        in_specs=[
            pl.BlockSpec(memory_space=pltpu.SMEM),
            pl.BlockSpec(memory_space=pltpu.VMEM),
            pl.BlockSpec(memory_space=pltpu.VMEM),
        ],
        out_specs=pl.BlockSpec(memory_space=pltpu.VMEM),
    )(rs, x_t, lab_t)


@jax.jit
def kernel(coords, tidx, rs):
    loss = _loss_call(coords.T, tidx.T, rs)
    return (coords, loss[0, 0])
